# baseline (speedup 1.0000x reference)
import functools
import jax
import jax.numpy as jnp
from jax import lax
from jax.experimental import pallas as pl
from jax.experimental.pallas import tpu as pltpu
from jax.experimental.pallas import tpu_sc as plsc

_B, _S, _D = 16, 4096, 768
_NW = 32
_HALF = _S // 2
_CR = 64
_NCHUNK = _HALF // _CR
_VPR = _D // 16


def _sc_body(x_hbm, mask_hbm, pos_hbm, out_hbm, xbuf, pbuf, mbuf):
    wid = lax.axis_index("s") * 2 + lax.axis_index("c")
    b = wid // 2
    h = wid % 2

    def chunk_body(c, _):
        row0 = h * _HALF + c * _CR
        n = _CR * _D
        pltpu.sync_copy(x_hbm.at[pl.ds(b * _S * _D + row0 * _D, n)], xbuf)
        pltpu.sync_copy(pos_hbm.at[pl.ds(row0 * _D, n)], pbuf)
        pltpu.sync_copy(mask_hbm.at[pl.ds(row0, _CR)], mbuf)

        def row_body(r, _):
            m = plsc.load_gather(mbuf, [jnp.full((16,), r, jnp.int32)])
            for k in range(_VPR):
                off = r * _D + k * 16
                xbuf[pl.ds(off, 16)] = xbuf[pl.ds(off, 16)] + pbuf[pl.ds(off, 16)] * m
            return 0

        lax.fori_loop(0, _CR, row_body, 0)
        pltpu.sync_copy(xbuf, out_hbm.at[pl.ds(b * _S * _D + row0 * _D, n)])
        return 0

    lax.fori_loop(0, _NCHUNK, chunk_body, 0)


def kernel(x, mask, pos_embed):
    B, S, D = x.shape
    x_flat = x.reshape(B * S * D)
    pos_flat = pos_embed.reshape(S * D)
    maskf = mask.reshape(S).astype(jnp.float32)
    mesh = plsc.VectorSubcoreMesh(core_axis_name="c", subcore_axis_name="s")
    sc = functools.partial(
        pl.kernel,
        mesh=mesh,
        out_type=jax.ShapeDtypeStruct((B * S * D,), jnp.float32),
        scratch_types=[
            pltpu.VMEM((_CR * _D,), jnp.float32),
            pltpu.VMEM((_CR * _D,), jnp.float32),
            pltpu.VMEM((_CR,), jnp.float32),
        ],
        compiler_params=pltpu.CompilerParams(needs_layout_passes=False),
    )(_sc_body)
    out = sc(x_flat, maskf, pos_flat)
    return out.reshape(B, S, D)


# speedup vs baseline: 9.5702x; 9.5702x over previous
import jax
import jax.numpy as jnp
from jax.experimental import pallas as pl
from jax.experimental.pallas import tpu as pltpu


_BLOCK_S = 4096


def _add_pos_kernel(x_ref, mask_ref, pos_ref, out_ref):
    m = mask_ref[...].astype(jnp.float32)
    out_ref[0] = x_ref[0] + pos_ref[0] * m


def kernel(x, mask, pos_embed):
    B, S, D = x.shape
    maskf = mask.reshape(S, 1)
    bs = _BLOCK_S
    grid = (S // bs, B)
    return pl.pallas_call(
        _add_pos_kernel,
        grid=grid,
        in_specs=[
            pl.BlockSpec((1, bs, D), lambda i, j: (j, i, 0)),
            pl.BlockSpec((bs, 1), lambda i, j: (i, 0)),
            pl.BlockSpec((1, bs, D), lambda i, j: (0, i, 0)),
        ],
        out_specs=pl.BlockSpec((1, bs, D), lambda i, j: (j, i, 0)),
        out_shape=jax.ShapeDtypeStruct((B, S, D), x.dtype),
        compiler_params=pltpu.CompilerParams(
            dimension_semantics=("parallel", "parallel"),
            vmem_limit_bytes=110 * 1024 * 1024,
        ),
    )(x, maskf, pos_embed)
